# baseline (speedup 1.0000x reference)
import functools

import jax
import jax.numpy as jnp
from jax import lax
from jax.experimental import pallas as pl
from jax.experimental.pallas import tpu as pltpu
from jax.experimental.pallas import tpu_sc as plsc

B = 4
HB = 2
N = 4096
NPOINT = 1024
NSAMPLE = 32
CH = 128
R2 = 0.2 * 0.2
BM = 128
NWORK = 32
PERW = B * NPOINT // NWORK


def _tc_front_body(pts_ref, xyzt_ref, nxt_ref, wp_ref, wx_ref, bias_ref,
                   f_ref, gidx_ref, ct_ref):
    bidx = pl.program_id(0)
    jidx = pl.program_id(1)
    xyzt = xyzt_ref[0]

    @pl.when(jidx == 0)
    def _():
        f = lax.dot_general(pts_ref[0], wp_ref[...], (((0,), (1,)), ((), ())),
                            preferred_element_type=jnp.float32)
        f_ref[0] = f + lax.dot_general(xyzt, wx_ref[...], (((0,), (1,)), ((), ())),
                                       preferred_element_type=jnp.float32)

    nx = nxt_ref[0]
    prod = lax.dot_general(nx, xyzt, (((0,), (0,)), ((), ())),
                           preferred_element_type=jnp.float32)
    cn = nx[0] * nx[0] + nx[1] * nx[1] + nx[2] * nx[2]
    pn = xyzt[0] * xyzt[0] + xyzt[1] * xyzt[1] + xyzt[2] * xyzt[2]
    d = -2.0 * prod
    d = d + cn[:, None]
    d = d + pn[None, :]
    iot = lax.broadcasted_iota(jnp.int32, (BM, N), 1).astype(jnp.float32)
    fbig = float(N)
    p = jnp.where(d > R2, fbig, iot)
    q0, q1 = p[:, :N // 4], p[:, N // 4:N // 2]
    q2, q3 = p[:, N // 2:3 * N // 4], p[:, 3 * N // 4:]
    lo1, hi1 = jnp.minimum(q0, q1), jnp.maximum(q0, q1)
    lo2, hi2 = jnp.minimum(q2, q3), jnp.maximum(q2, q3)
    c0, m1 = jnp.minimum(lo1, lo2), jnp.maximum(lo1, lo2)
    c3, m2 = jnp.maximum(hi1, hi2), jnp.minimum(hi1, hi2)
    c1, c2 = jnp.minimum(m1, m2), jnp.maximum(m1, m2)
    cols = []
    for _ in range(NSAMPLE):
        v = jnp.min(c0, axis=1, keepdims=True)
        cols.append(v)
        eq = c0 == v
        c0 = jnp.where(eq, c1, c0)
        c1 = jnp.where(eq, c2, c1)
        c2 = jnp.where(eq, c3, c2)
        c3 = jnp.where(eq, fbig, c3)
    g = jnp.concatenate(cols, axis=1).astype(jnp.int32)
    first = g[:, 0:1]
    first = jnp.where(first == N, 0, first)
    g = jnp.where(g == N, first, g)
    gidx_ref[0] = g + bidx * N
    ct = bias_ref[...] - lax.dot_general(nx, wx_ref[...], (((0,), (1,)), ((), ())),
                                         preferred_element_type=jnp.float32)
    ct_ref[0] = ct


QC = 4
NCHUNK = PERW // QC


def _gathermax_body(f_hbm, idx_hbm, out_hbm, idx_v, rows_v, out_v, sem0, sem1):
    wid = lax.axis_index("s") * 2 + lax.axis_index("c")
    pltpu.sync_copy(idx_hbm.at[pl.ds(wid * (PERW * NSAMPLE), PERW * NSAMPLE)], idx_v)

    def issue(k, buf, sem):
        pltpu.async_copy(f_hbm.at[idx_v.at[pl.ds(k * (QC * NSAMPLE), QC * NSAMPLE)]],
                         rows_v.at[buf], sem)

    def drain(buf, sem):
        pltpu.make_async_copy(f_hbm.at[idx_v.at[pl.ds(0, QC * NSAMPLE)]],
                              rows_v.at[buf], sem).wait()

    def compute(k, buf):
        for q in range(QC):
            for gi in range(CH // 16):
                acc = rows_v[buf, q * NSAMPLE, pl.ds(gi * 16, 16)]
                for s in range(1, NSAMPLE):
                    acc = jnp.maximum(acc, rows_v[buf, q * NSAMPLE + s, pl.ds(gi * 16, 16)])
                out_v[k * QC + q, pl.ds(gi * 16, 16)] = acc

    issue(0, 0, sem0)

    def body(k2, carry):
        k = k2 * 2
        issue(k + 1, 1, sem1)
        drain(0, sem0)
        compute(k, 0)

        @pl.when(k + 2 < NCHUNK)
        def _():
            issue(k + 2, 0, sem0)

        drain(1, sem1)
        compute(k + 1, 1)
        return carry

    lax.fori_loop(0, NCHUNK // 2, body, 0)
    pltpu.sync_copy(out_v, out_hbm.at[pl.ds(wid * PERW, PERW)])


def _bn_body(x0_ref, ct0_ref, gamma_ref, beta_ref, o_ref):
    x = x0_ref[...] + ct0_ref[...]
    m = jnp.mean(x, axis=0, keepdims=True)
    v = jnp.mean((x - m) ** 2, axis=0, keepdims=True)
    y = (x - m) / jnp.sqrt(v + 1e-5) * gamma_ref[...] + beta_ref[...]
    y = jnp.maximum(y, 0.0)
    for bb in range(B):
        o_ref[bb] = y[bb * NPOINT:(bb + 1) * NPOINT, :].T


def kernel(xyz, points, W, b, gamma, beta):
    xyzt = jnp.transpose(xyz, (0, 2, 1))
    nxt = xyzt[:, :, ::N // NPOINT]
    wx = W[:, :3]
    wp = W[:, 3:]

    def tc_front():
        return pl.pallas_call(
            _tc_front_body,
            grid=(B, NPOINT // BM),
            in_specs=[
                pl.BlockSpec((1, CH, N), lambda i, j: (i, 0, 0)),
                pl.BlockSpec((1, 3, N), lambda i, j: (i, 0, 0)),
                pl.BlockSpec((1, 3, BM), lambda i, j: (i, 0, j)),
                pl.BlockSpec((CH, CH), lambda i, j: (0, 0)),
                pl.BlockSpec((CH, 3), lambda i, j: (0, 0)),
                pl.BlockSpec((1, CH), lambda i, j: (0, 0)),
            ],
            out_specs=[
                pl.BlockSpec((1, N, CH), lambda i, j: (i, 0, 0)),
                pl.BlockSpec((1, BM, NSAMPLE), lambda i, j: (i, j, 0)),
                pl.BlockSpec((1, BM, CH), lambda i, j: (i, j, 0)),
            ],
            out_shape=[
                jax.ShapeDtypeStruct((B, N, CH), jnp.float32),
                jax.ShapeDtypeStruct((B, NPOINT, NSAMPLE), jnp.int32),
                jax.ShapeDtypeStruct((B, NPOINT, CH), jnp.float32),
            ],
        )(points, xyzt, nxt, wp, wx, b[None, :])

    gmax = pl.kernel(
        _gathermax_body,
        mesh=plsc.VectorSubcoreMesh(core_axis_name="c", subcore_axis_name="s"),
        out_type=jax.ShapeDtypeStruct((B * NPOINT, CH), jnp.float32),
        scratch_types=[
            pltpu.VMEM((PERW * NSAMPLE,), jnp.int32),
            pltpu.VMEM((2, QC * NSAMPLE, CH), jnp.float32),
            pltpu.VMEM((PERW, CH), jnp.float32),
            pltpu.SemaphoreType.DMA,
            pltpu.SemaphoreType.DMA,
        ],
    )

    f0, gidx0, ct0 = tc_front()
    out0 = gmax(f0.reshape(B * N, CH), gidx0.reshape(B * NPOINT * NSAMPLE))

    y = pl.pallas_call(
        _bn_body,
        in_specs=[
            pl.BlockSpec((B * NPOINT, CH), lambda: (0, 0)),
            pl.BlockSpec((B * NPOINT, CH), lambda: (0, 0)),
            pl.BlockSpec((1, CH), lambda: (0, 0)),
            pl.BlockSpec((1, CH), lambda: (0, 0)),
        ],
        out_specs=pl.BlockSpec((B, CH, NPOINT), lambda: (0, 0, 0)),
        out_shape=jax.ShapeDtypeStruct((B, CH, NPOINT), jnp.float32),
    )(out0, ct0.reshape(B * NPOINT, CH), gamma[None, :], beta[None, :])

    new_xyz = xyz[:, ::N // NPOINT, :]
    return (new_xyz, y)


# speedup vs baseline: 1.0901x; 1.0901x over previous
import functools

import jax
import jax.numpy as jnp
from jax import lax
from jax.experimental import pallas as pl
from jax.experimental.pallas import tpu as pltpu
from jax.experimental.pallas import tpu_sc as plsc

B = 4
HB = 2
N = 4096
NPOINT = 1024
NSAMPLE = 32
CH = 128
R2 = 0.2 * 0.2
BM = 128
NWORK = 32
PERW = HB * NPOINT // NWORK


def _tc_front_body(pts_ref, xyzt_ref, nxt_ref, wp_ref, wx_ref, bias_ref,
                   f_ref, gidx_ref, ct_ref):
    bidx = pl.program_id(0)
    jidx = pl.program_id(1)
    xyzt = xyzt_ref[0]

    @pl.when(jidx == 0)
    def _():
        f = lax.dot_general(pts_ref[0], wp_ref[...], (((0,), (1,)), ((), ())),
                            preferred_element_type=jnp.float32)
        f_ref[0] = f + lax.dot_general(xyzt, wx_ref[...], (((0,), (1,)), ((), ())),
                                       preferred_element_type=jnp.float32)

    nx = nxt_ref[0]
    prod = lax.dot_general(nx, xyzt, (((0,), (0,)), ((), ())),
                           preferred_element_type=jnp.float32)
    cn = nx[0] * nx[0] + nx[1] * nx[1] + nx[2] * nx[2]
    pn = xyzt[0] * xyzt[0] + xyzt[1] * xyzt[1] + xyzt[2] * xyzt[2]
    d = -2.0 * prod
    d = d + cn[:, None]
    d = d + pn[None, :]
    iot = lax.broadcasted_iota(jnp.int32, (BM, N), 1).astype(jnp.float32)
    fbig = float(N)
    p = jnp.where(d > R2, fbig, iot)
    c = [p[:, k * (N // 8):(k + 1) * (N // 8)] for k in range(8)]
    for a, bb2 in ((0, 1), (2, 3), (4, 5), (6, 7),
                   (0, 2), (1, 3), (4, 6), (5, 7),
                   (1, 2), (5, 6),
                   (0, 4), (1, 5), (2, 6), (3, 7),
                   (2, 4), (3, 5),
                   (1, 2), (3, 4), (5, 6)):
        lo = jnp.minimum(c[a], c[bb2])
        hi = jnp.maximum(c[a], c[bb2])
        c[a], c[bb2] = lo, hi
    cols = []
    for _ in range(NSAMPLE):
        v = jnp.min(c[0], axis=1, keepdims=True)
        cols.append(v)
        eq = c[0] == v
        for k in range(7):
            c[k] = jnp.where(eq, c[k + 1], c[k])
        c[7] = jnp.where(eq, fbig, c[7])
    g = jnp.concatenate(cols, axis=1).astype(jnp.int32)
    first = g[:, 0:1]
    first = jnp.where(first == N, 0, first)
    g = jnp.where(g == N, first, g)
    gidx_ref[0] = g + bidx * N
    ct = bias_ref[...] - lax.dot_general(nx, wx_ref[...], (((0,), (1,)), ((), ())),
                                         preferred_element_type=jnp.float32)
    ct_ref[0] = ct


QC = 4
NCHUNK = PERW // QC


def _gathermax_body(f_hbm, idx_hbm, out_hbm, idx_v, rows_v, out_v, sem0, sem1):
    wid = lax.axis_index("s") * 2 + lax.axis_index("c")
    pltpu.sync_copy(idx_hbm.at[pl.ds(wid * (PERW * NSAMPLE), PERW * NSAMPLE)], idx_v)

    def issue(k, buf, sem):
        pltpu.async_copy(f_hbm.at[idx_v.at[pl.ds(k * (QC * NSAMPLE), QC * NSAMPLE)]],
                         rows_v.at[buf], sem)

    def drain(buf, sem):
        pltpu.make_async_copy(f_hbm.at[idx_v.at[pl.ds(0, QC * NSAMPLE)]],
                              rows_v.at[buf], sem).wait()

    def compute(k, buf):
        for q in range(QC):
            for gi in range(CH // 16):
                acc = rows_v[buf, q * NSAMPLE, pl.ds(gi * 16, 16)]
                for s in range(1, NSAMPLE):
                    acc = jnp.maximum(acc, rows_v[buf, q * NSAMPLE + s, pl.ds(gi * 16, 16)])
                out_v[k * QC + q, pl.ds(gi * 16, 16)] = acc

    issue(0, 0, sem0)

    def body(k2, carry):
        k = k2 * 2
        issue(k + 1, 1, sem1)
        drain(0, sem0)
        compute(k, 0)

        @pl.when(k + 2 < NCHUNK)
        def _():
            issue(k + 2, 0, sem0)

        drain(1, sem1)
        compute(k + 1, 1)
        return carry

    lax.fori_loop(0, NCHUNK // 2, body, 0)
    pltpu.sync_copy(out_v, out_hbm.at[pl.ds(wid * PERW, PERW)])


def _bn_body(x0_ref, x1_ref, ct0_ref, ct1_ref, gamma_ref, beta_ref, o_ref):
    x = jnp.concatenate(
        [x0_ref[...] + ct0_ref[...], x1_ref[...] + ct1_ref[...]], axis=0)
    m = jnp.mean(x, axis=0, keepdims=True)
    v = jnp.mean((x - m) ** 2, axis=0, keepdims=True)
    y = (x - m) / jnp.sqrt(v + 1e-5) * gamma_ref[...] + beta_ref[...]
    y = jnp.maximum(y, 0.0)
    for bb in range(B):
        o_ref[bb] = y[bb * NPOINT:(bb + 1) * NPOINT, :].T


def kernel(xyz, points, W, b, gamma, beta):
    xyzt = jnp.transpose(xyz, (0, 2, 1))
    nxt = xyzt[:, :, ::N // NPOINT]
    wx = W[:, :3]
    wp = W[:, 3:]

    def tc_front(half):
        return pl.pallas_call(
            _tc_front_body,
            grid=(HB, NPOINT // BM),
            in_specs=[
                pl.BlockSpec((1, CH, N), lambda i, j: (half * HB + i, 0, 0)),
                pl.BlockSpec((1, 3, N), lambda i, j: (half * HB + i, 0, 0)),
                pl.BlockSpec((1, 3, BM), lambda i, j: (half * HB + i, 0, j)),
                pl.BlockSpec((CH, CH), lambda i, j: (0, 0)),
                pl.BlockSpec((CH, 3), lambda i, j: (0, 0)),
                pl.BlockSpec((1, CH), lambda i, j: (0, 0)),
            ],
            out_specs=[
                pl.BlockSpec((1, N, CH), lambda i, j: (i, 0, 0)),
                pl.BlockSpec((1, BM, NSAMPLE), lambda i, j: (i, j, 0)),
                pl.BlockSpec((1, BM, CH), lambda i, j: (i, j, 0)),
            ],
            out_shape=[
                jax.ShapeDtypeStruct((HB, N, CH), jnp.float32),
                jax.ShapeDtypeStruct((HB, NPOINT, NSAMPLE), jnp.int32),
                jax.ShapeDtypeStruct((HB, NPOINT, CH), jnp.float32),
            ],
        )(points, xyzt, nxt, wp, wx, b[None, :])

    gmax = pl.kernel(
        _gathermax_body,
        mesh=plsc.VectorSubcoreMesh(core_axis_name="c", subcore_axis_name="s"),
        out_type=jax.ShapeDtypeStruct((HB * NPOINT, CH), jnp.float32),
        scratch_types=[
            pltpu.VMEM((PERW * NSAMPLE,), jnp.int32),
            pltpu.VMEM((2, QC * NSAMPLE, CH), jnp.float32),
            pltpu.VMEM((PERW, CH), jnp.float32),
            pltpu.SemaphoreType.DMA,
            pltpu.SemaphoreType.DMA,
        ],
    )

    f0, gidx0, ct0 = tc_front(0)
    f1, gidx1, ct1 = tc_front(1)
    out0 = gmax(f0.reshape(HB * N, CH), gidx0.reshape(HB * NPOINT * NSAMPLE))
    out1 = gmax(f1.reshape(HB * N, CH), gidx1.reshape(HB * NPOINT * NSAMPLE))

    y = pl.pallas_call(
        _bn_body,
        in_specs=[
            pl.BlockSpec((HB * NPOINT, CH), lambda: (0, 0)),
            pl.BlockSpec((HB * NPOINT, CH), lambda: (0, 0)),
            pl.BlockSpec((HB * NPOINT, CH), lambda: (0, 0)),
            pl.BlockSpec((HB * NPOINT, CH), lambda: (0, 0)),
            pl.BlockSpec((1, CH), lambda: (0, 0)),
            pl.BlockSpec((1, CH), lambda: (0, 0)),
        ],
        out_specs=pl.BlockSpec((B, CH, NPOINT), lambda: (0, 0, 0)),
        out_shape=jax.ShapeDtypeStruct((B, CH, NPOINT), jnp.float32),
    )(out0, out1, ct0.reshape(HB * NPOINT, CH), ct1.reshape(HB * NPOINT, CH),
      gamma[None, :], beta[None, :])

    new_xyz = xyz[:, ::N // NPOINT, :]
    return (new_xyz, y)


# speedup vs baseline: 1.6767x; 1.5381x over previous
import functools

import jax
import jax.numpy as jnp
from jax import lax
from jax.experimental import pallas as pl
from jax.experimental.pallas import tpu as pltpu
from jax.experimental.pallas import tpu_sc as plsc

B = 4
HB = 2
N = 4096
NPOINT = 1024
NSAMPLE = 32
CH = 128
R2 = 0.2 * 0.2
BM = 128
NWORK = 32
PERW = HB * NPOINT // NWORK


def _tc_front_body(pts_ref, xyzt_ref, nxt_ref, wp_ref, wx_ref, bias_ref,
                   f_ref, gidx_ref, ct_ref):
    bidx = pl.program_id(0)
    jidx = pl.program_id(1)
    xyzt = xyzt_ref[0]

    @pl.when(jidx == 0)
    def _():
        f = lax.dot_general(pts_ref[0], wp_ref[...], (((0,), (1,)), ((), ())),
                            preferred_element_type=jnp.float32)
        f_ref[0] = f + lax.dot_general(xyzt, wx_ref[...], (((0,), (1,)), ((), ())),
                                       preferred_element_type=jnp.float32)

    nx = nxt_ref[0]
    prod = lax.dot_general(nx, xyzt, (((0,), (0,)), ((), ())),
                           preferred_element_type=jnp.float32)
    cn = nx[0] * nx[0] + nx[1] * nx[1] + nx[2] * nx[2]
    pn = xyzt[0] * xyzt[0] + xyzt[1] * xyzt[1] + xyzt[2] * xyzt[2]
    d = -2.0 * prod
    d = d + cn[:, None]
    d = d + pn[None, :]
    SG = N // 8
    s = jnp.full((BM, SG), -1, jnp.int32)
    for k in range(7, -1, -1):
        mk = jnp.logical_not(d[:, k * SG:(k + 1) * SG] > R2)
        s = jnp.where(mk, (s << 4) | k, s)
    iotj = lax.broadcasted_iota(jnp.int32, (BM, SG), 1).astype(jnp.float32)
    sentinel_top = jnp.int32(-268435456)
    cols = []
    for _ in range(NSAMPLE):
        val = iotj + (s & 15).astype(jnp.float32) * float(SG)
        v = jnp.min(val, axis=1, keepdims=True)
        cols.append(v)
        eq = val == v
        s = jnp.where(eq, (s >> 4) | sentinel_top, s)
    g = jnp.concatenate(cols, axis=1).astype(jnp.int32)
    g = jnp.minimum(g, N)
    first = g[:, 0:1]
    first = jnp.where(first == N, 0, first)
    g = jnp.where(g == N, first, g)
    gidx_ref[0] = g + bidx * N
    ct = bias_ref[...] - lax.dot_general(nx, wx_ref[...], (((0,), (1,)), ((), ())),
                                         preferred_element_type=jnp.float32)
    ct_ref[0] = ct


QC = 4
NCHUNK = PERW // QC


def _gathermax_body(f_hbm, idx_hbm, out_hbm, idx_v, rows_v, out_v, sem0, sem1):
    wid = lax.axis_index("s") * 2 + lax.axis_index("c")
    pltpu.sync_copy(idx_hbm.at[pl.ds(wid * (PERW * NSAMPLE), PERW * NSAMPLE)], idx_v)

    def issue(k, buf, sem):
        pltpu.async_copy(f_hbm.at[idx_v.at[pl.ds(k * (QC * NSAMPLE), QC * NSAMPLE)]],
                         rows_v.at[buf], sem)

    def drain(buf, sem):
        pltpu.make_async_copy(f_hbm.at[idx_v.at[pl.ds(0, QC * NSAMPLE)]],
                              rows_v.at[buf], sem).wait()

    def compute(k, buf):
        for q in range(QC):
            for gi in range(CH // 16):
                acc = rows_v[buf, q * NSAMPLE, pl.ds(gi * 16, 16)]
                for s in range(1, NSAMPLE):
                    acc = jnp.maximum(acc, rows_v[buf, q * NSAMPLE + s, pl.ds(gi * 16, 16)])
                out_v[k * QC + q, pl.ds(gi * 16, 16)] = acc

    issue(0, 0, sem0)

    def body(k2, carry):
        k = k2 * 2
        issue(k + 1, 1, sem1)
        drain(0, sem0)
        compute(k, 0)

        @pl.when(k + 2 < NCHUNK)
        def _():
            issue(k + 2, 0, sem0)

        drain(1, sem1)
        compute(k + 1, 1)
        return carry

    lax.fori_loop(0, NCHUNK // 2, body, 0)
    pltpu.sync_copy(out_v, out_hbm.at[pl.ds(wid * PERW, PERW)])


def _bn_body(x0_ref, x1_ref, ct0_ref, ct1_ref, gamma_ref, beta_ref, o_ref):
    x = jnp.concatenate(
        [x0_ref[...] + ct0_ref[...], x1_ref[...] + ct1_ref[...]], axis=0)
    m = jnp.mean(x, axis=0, keepdims=True)
    v = jnp.mean((x - m) ** 2, axis=0, keepdims=True)
    y = (x - m) / jnp.sqrt(v + 1e-5) * gamma_ref[...] + beta_ref[...]
    y = jnp.maximum(y, 0.0)
    for bb in range(B):
        o_ref[bb] = y[bb * NPOINT:(bb + 1) * NPOINT, :].T


def kernel(xyz, points, W, b, gamma, beta):
    xyzt = jnp.transpose(xyz, (0, 2, 1))
    nxt = xyzt[:, :, ::N // NPOINT]
    wx = W[:, :3]
    wp = W[:, 3:]

    def tc_front(half):
        return pl.pallas_call(
            _tc_front_body,
            grid=(HB, NPOINT // BM),
            in_specs=[
                pl.BlockSpec((1, CH, N), lambda i, j: (half * HB + i, 0, 0)),
                pl.BlockSpec((1, 3, N), lambda i, j: (half * HB + i, 0, 0)),
                pl.BlockSpec((1, 3, BM), lambda i, j: (half * HB + i, 0, j)),
                pl.BlockSpec((CH, CH), lambda i, j: (0, 0)),
                pl.BlockSpec((CH, 3), lambda i, j: (0, 0)),
                pl.BlockSpec((1, CH), lambda i, j: (0, 0)),
            ],
            out_specs=[
                pl.BlockSpec((1, N, CH), lambda i, j: (i, 0, 0)),
                pl.BlockSpec((1, BM, NSAMPLE), lambda i, j: (i, j, 0)),
                pl.BlockSpec((1, BM, CH), lambda i, j: (i, j, 0)),
            ],
            out_shape=[
                jax.ShapeDtypeStruct((HB, N, CH), jnp.float32),
                jax.ShapeDtypeStruct((HB, NPOINT, NSAMPLE), jnp.int32),
                jax.ShapeDtypeStruct((HB, NPOINT, CH), jnp.float32),
            ],
        )(points, xyzt, nxt, wp, wx, b[None, :])

    gmax = pl.kernel(
        _gathermax_body,
        mesh=plsc.VectorSubcoreMesh(core_axis_name="c", subcore_axis_name="s"),
        out_type=jax.ShapeDtypeStruct((HB * NPOINT, CH), jnp.float32),
        scratch_types=[
            pltpu.VMEM((PERW * NSAMPLE,), jnp.int32),
            pltpu.VMEM((2, QC * NSAMPLE, CH), jnp.float32),
            pltpu.VMEM((PERW, CH), jnp.float32),
            pltpu.SemaphoreType.DMA,
            pltpu.SemaphoreType.DMA,
        ],
    )

    f0, gidx0, ct0 = tc_front(0)
    f1, gidx1, ct1 = tc_front(1)
    out0 = gmax(f0.reshape(HB * N, CH), gidx0.reshape(HB * NPOINT * NSAMPLE))
    out1 = gmax(f1.reshape(HB * N, CH), gidx1.reshape(HB * NPOINT * NSAMPLE))

    y = pl.pallas_call(
        _bn_body,
        in_specs=[
            pl.BlockSpec((HB * NPOINT, CH), lambda: (0, 0)),
            pl.BlockSpec((HB * NPOINT, CH), lambda: (0, 0)),
            pl.BlockSpec((HB * NPOINT, CH), lambda: (0, 0)),
            pl.BlockSpec((HB * NPOINT, CH), lambda: (0, 0)),
            pl.BlockSpec((1, CH), lambda: (0, 0)),
            pl.BlockSpec((1, CH), lambda: (0, 0)),
        ],
        out_specs=pl.BlockSpec((B, CH, NPOINT), lambda: (0, 0, 0)),
        out_shape=jax.ShapeDtypeStruct((B, CH, NPOINT), jnp.float32),
    )(out0, out1, ct0.reshape(HB * NPOINT, CH), ct1.reshape(HB * NPOINT, CH),
      gamma[None, :], beta[None, :])

    new_xyz = xyz[:, ::N // NPOINT, :]
    return (new_xyz, y)
